# baseline (speedup 1.0000x reference)
import functools

import jax
import jax.numpy as jnp
from jax import lax
from jax.experimental import pallas as pl
from jax.experimental.pallas import tpu as pltpu
from jax.experimental.pallas import tpu_sc as plsc

_ROWS = 262144
_DIM = 128
_NC = 2
_NS = 16
_NW = _NC * _NS
_ROWS_W = _ROWS // _NW
_HALF = _ROWS_W // 2
_CHUNK = 256
_NBUF = 2
_NITER = _HALF // _CHUNK
_NGROUPS = _NITER // _NBUF


def _sc_copy_body(in_hbm, out_hbm, abuf, bbuf, a_in, a_out, b_in, b_out):
    cid = lax.axis_index("c")
    sid = lax.axis_index("s")
    wid = sid * _NC + cid
    base_a = wid * _ROWS_W
    base_b = base_a + _HALF

    def a_in_cp(row, b):
        return pltpu.make_async_copy(
            in_hbm.at[pl.ds(row, _CHUNK), :], abuf.at[b], a_in.at[b]
        )

    def a_out_cp(row, b):
        return pltpu.make_async_copy(
            abuf.at[b], out_hbm.at[pl.ds(row, _CHUNK), :], a_out.at[b]
        )

    def b_in_cp(row, b):
        return pltpu.make_async_copy(
            in_hbm.at[pl.ds(row, _CHUNK), :], bbuf.at[sid, b], b_in.at[b]
        )

    def b_out_cp(row, b):
        return pltpu.make_async_copy(
            bbuf.at[sid, b], out_hbm.at[pl.ds(row, _CHUNK), :], b_out.at[b]
        )

    for b in range(_NBUF):
        a_in_cp(base_a + b * _CHUNK, b).start()
        b_in_cp(base_b + b * _CHUNK, b).start()

    def step(i_dyn, b, prefetch):
        row_a = base_a + i_dyn * _CHUNK
        row_b = base_b + i_dyn * _CHUNK
        a_in_cp(row_a, b).wait()
        a_out_cp(row_a, b).start()
        b_in_cp(row_b, b).wait()
        b_out_cp(row_b, b).start()
        a_out_cp(row_a, b).wait()
        if prefetch:
            a_in_cp(row_a + _NBUF * _CHUNK, b).start()
        b_out_cp(row_b, b).wait()
        if prefetch:
            b_in_cp(row_b + _NBUF * _CHUNK, b).start()

    def group(g, carry):
        for b in range(_NBUF):
            step(g * _NBUF + b, b, prefetch=True)
        return carry

    lax.fori_loop(0, _NGROUPS - 1, group, 0)

    for b in range(_NBUF):
        step((_NGROUPS - 1) * _NBUF + b, b, prefetch=False)


_sc_copy = functools.partial(
    pl.kernel,
    out_type=jax.ShapeDtypeStruct((_ROWS, _DIM), jnp.float32),
    mesh=plsc.VectorSubcoreMesh(core_axis_name="c", subcore_axis_name="s"),
    scratch_types=[
        pltpu.VMEM((_NBUF, _CHUNK, _DIM), jnp.float32),
        pltpu.VMEM_SHARED((_NS, _NBUF, _CHUNK, _DIM), jnp.float32),
        pltpu.SemaphoreType.DMA((_NBUF,)),
        pltpu.SemaphoreType.DMA((_NBUF,)),
        pltpu.SemaphoreType.DMA((_NBUF,)),
        pltpu.SemaphoreType.DMA((_NBUF,)),
    ],
)(_sc_copy_body)


def kernel(queue):
    return _sc_copy(queue)


# speedup vs baseline: 1.2804x; 1.2804x over previous
import jax
import jax.numpy as jnp
from jax.experimental import pallas as pl
from jax.experimental.pallas import tpu as pltpu

_BLK = 16384


def _copy_body(in_ref, out_ref):
    out_ref[...] = in_ref[...]


def kernel(queue):
    rows, dim = queue.shape
    return pl.pallas_call(
        _copy_body,
        out_shape=jax.ShapeDtypeStruct(queue.shape, queue.dtype),
        grid=(rows // _BLK,),
        in_specs=[pl.BlockSpec((_BLK, dim), lambda i: (i, 0))],
        out_specs=pl.BlockSpec((_BLK, dim), lambda i: (i, 0)),
        compiler_params=pltpu.CompilerParams(
            dimension_semantics=("parallel",),
        ),
    )(queue)
